# Initial kernel scaffold; baseline (speedup 1.0000x reference)
import functools

import jax
import jax.numpy as jnp
from jax import lax
from jax.experimental import pallas as pl
from jax.experimental.pallas import tpu as pltpu
from jax.experimental.pallas import tpu_sc as plsc

N = 10000
E = 320000
D = 128

NC = 2
NS = 16
CHUNK = 128
CHUNKS_PER_TILE = 80
E_PAD = NC * NS * CHUNKS_PER_TILE * CHUNK
N_PAD = 10240
ROWS_PER_TILE = N_PAD // NS

_VMESH = plsc.VectorSubcoreMesh(core_axis_name="c", subcore_axis_name="s")


@functools.partial(
    pl.kernel,
    out_type=jax.ShapeDtypeStruct((NC, N_PAD, D), jnp.float32),
    mesh=_VMESH,
    scratch_types=[
        pltpu.VMEM((CHUNK,), jnp.int32),
        pltpu.VMEM((CHUNK,), jnp.int32),
        pltpu.VMEM((CHUNK, D), jnp.float32),
        pltpu.VMEM((64, D), jnp.float32),
        pltpu.VMEM_SHARED((N_PAD, D), jnp.float32),
        pltpu.SemaphoreType.DMA,
    ],
)
def _sc_agg(h_hbm, src_hbm, dst_hbm, out_hbm, src_v, dst_v, rows_v, zero_v,
            acc_sh, sem):
    cid = lax.axis_index("c")
    sid = lax.axis_index("s")

    @pl.loop(0, 64)
    def _(i):
        @pl.loop(0, D // 16)
        def _(j):
            zero_v[i, pl.ds(j * 16, 16)] = jnp.zeros((16,), jnp.float32)

    row0 = sid * ROWS_PER_TILE

    @pl.loop(0, ROWS_PER_TILE // 64)
    def _(i):
        pltpu.sync_copy(zero_v, acc_sh.at[pl.ds(row0 + i * 64, 64)])

    plsc.subcore_barrier()

    base_chunk = (cid * NS + sid) * CHUNKS_PER_TILE

    @pl.loop(0, CHUNKS_PER_TILE)
    def _(i):
        off = (base_chunk + i) * CHUNK
        pltpu.sync_copy(src_hbm.at[pl.ds(off, CHUNK)], src_v)
        pltpu.sync_copy(dst_hbm.at[pl.ds(off, CHUNK)], dst_v)
        pltpu.async_copy(h_hbm.at[src_v], rows_v, sem).wait()
        pltpu.sync_copy(rows_v, acc_sh.at[dst_v], add=True)

    plsc.subcore_barrier()
    pltpu.sync_copy(acc_sh.at[pl.ds(row0, ROWS_PER_TILE)],
                    out_hbm.at[cid].at[pl.ds(row0, ROWS_PER_TILE)])


@functools.partial(
    pl.kernel,
    out_type=jax.ShapeDtypeStruct((NC, N_PAD, 16), jnp.float32),
    mesh=_VMESH,
    scratch_types=[
        pltpu.VMEM((CHUNK,), jnp.int32),
        pltpu.VMEM((CHUNK, 16), jnp.float32),
        pltpu.VMEM((64, 16), jnp.float32),
        pltpu.VMEM_SHARED((N_PAD, 16), jnp.float32),
    ],
)
def _sc_count(dst_hbm, out_hbm, dst_v, ones_v, zero_v, acc_sh):
    cid = lax.axis_index("c")
    sid = lax.axis_index("s")

    @pl.loop(0, CHUNK)
    def _(i):
        ones_v[i] = jnp.ones((16,), jnp.float32)

    @pl.loop(0, 64)
    def _(i):
        zero_v[i] = jnp.zeros((16,), jnp.float32)

    row0 = sid * ROWS_PER_TILE

    @pl.loop(0, ROWS_PER_TILE // 64)
    def _(i):
        pltpu.sync_copy(zero_v, acc_sh.at[pl.ds(row0 + i * 64, 64)])

    plsc.subcore_barrier()

    base_chunk = (cid * NS + sid) * CHUNKS_PER_TILE

    @pl.loop(0, CHUNKS_PER_TILE)
    def _(i):
        off = (base_chunk + i) * CHUNK
        pltpu.sync_copy(dst_hbm.at[pl.ds(off, CHUNK)], dst_v)
        pltpu.sync_copy(ones_v, acc_sh.at[dst_v], add=True)

    plsc.subcore_barrier()
    pltpu.sync_copy(acc_sh.at[pl.ds(row0, ROWS_PER_TILE)],
                    out_hbm.at[cid].at[pl.ds(row0, ROWS_PER_TILE)])


def _dot(a, b):
    return jax.lax.dot_general(a, b, (((1,), (0,)), ((), ())),
                               precision=jax.lax.Precision.HIGHEST,
                               preferred_element_type=jnp.float32)


def _inv_body(cnt_ref, inv_ref):
    c = cnt_ref[0, :N, 0:1] + cnt_ref[1, :N, 0:1]
    inv_ref[...] = 1.0 / jnp.maximum(c, 1.0)


_inv_call = pl.pallas_call(
    _inv_body, out_shape=jax.ShapeDtypeStruct((N, 1), jnp.float32))


def _enc_body(x_ref, w_ref, b_ref, out_ref):
    out_ref[...] = jnp.maximum(_dot(x_ref[...], w_ref[...]) + b_ref[...], 0.0)


_enc_call = pl.pallas_call(
    _enc_body, out_shape=jax.ShapeDtypeStruct((N, D), jnp.float32))


def _combine_body(agg_ref, inv_ref, h_ref, wl_ref, bl_ref, wr_ref, g_ref,
                  be_ref, out_ref):
    agg = (agg_ref[0, :N, :] + agg_ref[1, :N, :]) * inv_ref[...]
    h = h_ref[...]
    out = _dot(agg, wl_ref[...]) + bl_ref[...] + _dot(h, wr_ref[...])
    nrm = jnp.maximum(jnp.sqrt(jnp.sum(out * out, axis=1, keepdims=True)),
                      1e-12)
    out = out / nrm
    mean = jnp.mean(out, axis=0, keepdims=True)
    var = jnp.mean((out - mean) ** 2, axis=0, keepdims=True)
    out = (out - mean) * jax.lax.rsqrt(var + 1e-5) * g_ref[...] + be_ref[...]
    out_ref[...] = jnp.maximum(out, 0.0)


_combine_call = pl.pallas_call(
    _combine_body, out_shape=jax.ShapeDtypeStruct((N, D), jnp.float32))


def _heads_body(h_ref, wo1_ref, bo1_ref, wo2_ref, bo2_ref, wc1_ref, bc1_ref,
                wc2_ref, bc2_ref, wb1_ref, bb1_ref, wb2_ref, bb2_ref,
                order_ref, cost_ref, bull_ref):
    h = h_ref[...]
    order_ref[...] = _dot(jnp.maximum(_dot(h, wo1_ref[...]) + bo1_ref[...],
                                      0.0), wo2_ref[...]) + bo2_ref[...]
    cost_ref[...] = _dot(jnp.maximum(_dot(h, wc1_ref[...]) + bc1_ref[...],
                                     0.0), wc2_ref[...]) + bc2_ref[...]
    bull_ref[...] = _dot(jnp.maximum(_dot(h, wb1_ref[...]) + bb1_ref[...],
                                     0.0), wb2_ref[...]) + bb2_ref[...]


_heads_call = pl.pallas_call(
    _heads_body,
    out_shape=(jax.ShapeDtypeStruct((N, 1), jnp.float32),
               jax.ShapeDtypeStruct((N, 1), jnp.float32),
               jax.ShapeDtypeStruct((N, 1), jnp.float32)))


def kernel(x, edge_index, params):
    p = params
    pad = E_PAD - E
    src_p = jnp.concatenate(
        [edge_index[0], jnp.zeros((pad,), jnp.int32)])
    dst_p = jnp.concatenate(
        [edge_index[1], jnp.full((pad,), N_PAD - 1, jnp.int32)])

    cnt_parts = _sc_count(dst_p)
    inv = _inv_call(cnt_parts)
    h = _enc_call(x, p['W_enc'], p['b_enc'])
    for l in range(3):
        agg_parts = _sc_agg(h, src_p, dst_p)
        h = _combine_call(agg_parts, inv, h, p['W_l%d' % l], p['b_l%d' % l],
                          p['W_r%d' % l], p['bn_g%d' % l], p['bn_b%d' % l])
    order, cost, bull = _heads_call(
        h, p['W_o1'], p['b_o1'], p['W_o2'], p['b_o2'],
        p['W_c1'], p['b_c1'], p['W_c2'], p['b_c2'],
        p['W_b1'], p['b_b1'], p['W_b2'], p['b_b2'])
    return order, cost, bull, h


# speedup vs baseline: 3.3206x; 3.3206x over previous
import functools

import jax
import jax.numpy as jnp
from jax import lax
from jax.experimental import pallas as pl
from jax.experimental.pallas import tpu as pltpu
from jax.experimental.pallas import tpu_sc as plsc

N = 10000
E = 320000
D = 128

NC = 2
NS = 16
CHUNK = 128
CHUNKS_PER_TILE = 80
E_PAD = NC * NS * CHUNKS_PER_TILE * CHUNK
N_PAD = 10240
ROWS_PER_TILE = N_PAD // NS


TOTAL_CHUNKS = E_PAD // CHUNK
CPT = CHUNKS_PER_TILE


@functools.lru_cache(maxsize=None)
def _make_sc_agg():
    mesh = plsc.VectorSubcoreMesh(core_axis_name="c", subcore_axis_name="s")

    @functools.partial(
        pl.kernel,
        out_type=jax.ShapeDtypeStruct((NC, N_PAD, D), jnp.float32),
        mesh=mesh,
        scratch_types=[
            pltpu.VMEM((CHUNK,), jnp.int32),
            pltpu.VMEM((CHUNK,), jnp.int32),
            pltpu.VMEM((CHUNK,), jnp.int32),
            pltpu.VMEM((CHUNK,), jnp.int32),
            pltpu.VMEM((CHUNK, D), jnp.float32),
            pltpu.VMEM((CHUNK, D), jnp.float32),
            pltpu.VMEM((64, D), jnp.float32),
            pltpu.VMEM_SHARED((N_PAD, D), jnp.float32),
            pltpu.SemaphoreType.DMA,
            pltpu.SemaphoreType.DMA,
            pltpu.SemaphoreType.DMA,
            pltpu.SemaphoreType.DMA,
        ],
    )
    def sc_agg(h_hbm, src_hbm, dst_hbm, out_hbm, src_v0, src_v1, dst_v0,
               dst_v1, rows_v0, rows_v1, zero_v, acc_sh, sem_g0, sem_g1,
               sem_i0, sem_i1):
        cid = lax.axis_index("c")
        sid = lax.axis_index("s")
        base_chunk = (cid * NS + sid) * CPT

        @pl.loop(0, 64)
        def _(i):
            @pl.loop(0, D // 16)
            def _(j):
                zero_v[i, pl.ds(j * 16, 16)] = jnp.zeros((16,), jnp.float32)

        row0 = sid * ROWS_PER_TILE

        @pl.loop(0, ROWS_PER_TILE // 64)
        def _(i):
            pltpu.sync_copy(zero_v, acc_sh.at[pl.ds(row0 + i * 64, 64)])

        plsc.subcore_barrier()

        def idx(c, src_b, dst_b, sem):
            off = (base_chunk + c) * CHUNK
            return (pltpu.make_async_copy(src_hbm.at[pl.ds(off, CHUNK)],
                                          src_b, sem),
                    pltpu.make_async_copy(dst_hbm.at[pl.ds(off, CHUNK)],
                                          dst_b, sem))

        def gather(src_b, rows, sem):
            return pltpu.make_async_copy(h_hbm.at[src_b], rows, sem)

        a, b = idx(0, src_v0, dst_v0, sem_i0)
        a.start(); b.start(); a.wait(); b.wait()
        gather(src_v0, rows_v0, sem_g0).start()
        a, b = idx(1, src_v1, dst_v1, sem_i1)
        a.start(); b.start()

        @pl.loop(0, CPT // 2)
        def _(j):
            c0 = 2 * j
            c2 = jnp.minimum(c0 + 2, CPT - 1)
            c3 = jnp.minimum(c0 + 3, CPT - 1)
            a1, b1 = idx(c0 + 1, src_v1, dst_v1, sem_i1)
            a1.wait(); b1.wait()
            gather(src_v0, rows_v0, sem_g0).wait()
            gather(src_v1, rows_v1, sem_g1).start()
            pltpu.sync_copy(rows_v0, acc_sh.at[dst_v0], add=True)
            a2, b2 = idx(c2, src_v0, dst_v0, sem_i0)
            a2.start(); b2.start()
            gather(src_v1, rows_v1, sem_g1).wait()
            a2.wait(); b2.wait()
            gather(src_v0, rows_v0, sem_g0).start()
            pltpu.sync_copy(rows_v1, acc_sh.at[dst_v1], add=True)
            a3, b3 = idx(c3, src_v1, dst_v1, sem_i1)
            a3.start(); b3.start()

        gather(src_v0, rows_v0, sem_g0).wait()
        a, b = idx(CPT - 1, src_v1, dst_v1, sem_i1)
        a.wait(); b.wait()

        plsc.subcore_barrier()
        pltpu.sync_copy(acc_sh.at[pl.ds(row0, ROWS_PER_TILE)],
                        out_hbm.at[cid].at[pl.ds(row0, ROWS_PER_TILE)])

    return sc_agg


@functools.lru_cache(maxsize=None)
def _make_sc_count():
    mesh = plsc.VectorSubcoreMesh(core_axis_name="c", subcore_axis_name="s")

    @functools.partial(
        pl.kernel,
        out_type=jax.ShapeDtypeStruct((NC, N_PAD, D), jnp.float32),
        mesh=mesh,
        scratch_types=[
            pltpu.VMEM((CHUNK,), jnp.int32),
            pltpu.VMEM((CHUNK, D), jnp.float32),
            pltpu.VMEM((64, D), jnp.float32),
            pltpu.VMEM_SHARED((N_PAD, D), jnp.float32),
        ],
    )
    def sc_count(dst_hbm, out_hbm, dst_v, ones_v, zero_v, acc_sh):
        cid = lax.axis_index("c")
        sid = lax.axis_index("s")
        base_chunk = (cid * NS + sid) * CPT

        @pl.loop(0, CHUNK)
        def _(i):
            @pl.loop(0, D // 16)
            def _(j):
                ones_v[i, pl.ds(j * 16, 16)] = jnp.ones((16,), jnp.float32)

        @pl.loop(0, 64)
        def _(i):
            @pl.loop(0, D // 16)
            def _(j):
                zero_v[i, pl.ds(j * 16, 16)] = jnp.zeros((16,), jnp.float32)

        row0 = sid * ROWS_PER_TILE

        @pl.loop(0, ROWS_PER_TILE // 64)
        def _(i):
            pltpu.sync_copy(zero_v, acc_sh.at[pl.ds(row0 + i * 64, 64)])

        plsc.subcore_barrier()

        @pl.loop(0, CPT)
        def _(i):
            off = (base_chunk + i) * CHUNK
            pltpu.sync_copy(dst_hbm.at[pl.ds(off, CHUNK)], dst_v)
            pltpu.sync_copy(ones_v, acc_sh.at[dst_v], add=True)

        plsc.subcore_barrier()
        pltpu.sync_copy(acc_sh.at[pl.ds(row0, ROWS_PER_TILE)],
                        out_hbm.at[cid].at[pl.ds(row0, ROWS_PER_TILE)])

    return sc_count


_TC_PARAMS = pltpu.CompilerParams(vmem_limit_bytes=63 * 1024 * 1024)


def _dot(a, b):
    return jax.lax.dot_general(a, b, (((1,), (0,)), ((), ())),
                               preferred_element_type=jnp.float32)


def _inv_body(cnt_ref, inv_ref):
    c = cnt_ref[0, :N, 0:1] + cnt_ref[1, :N, 0:1]
    inv_ref[...] = 1.0 / jnp.maximum(c, 1.0)


_inv_call = pl.pallas_call(
    _inv_body, out_shape=jax.ShapeDtypeStruct((N, 1), jnp.float32),
    compiler_params=_TC_PARAMS)


def _enc_body(x_ref, w_ref, b_ref, out_ref):
    out_ref[...] = jnp.maximum(_dot(x_ref[...], w_ref[...]) + b_ref[...], 0.0)


_enc_call = pl.pallas_call(
    _enc_body, out_shape=jax.ShapeDtypeStruct((N, D), jnp.float32),
    compiler_params=_TC_PARAMS)


def _combine_body(agg_ref, inv_ref, h_ref, wl_ref, bl_ref, wr_ref, g_ref,
                  be_ref, out_ref):
    agg = (agg_ref[0, :N, :] + agg_ref[1, :N, :]) * inv_ref[...]
    h = h_ref[...]
    out = _dot(agg, wl_ref[...]) + bl_ref[...] + _dot(h, wr_ref[...])
    nrm = jnp.maximum(jnp.sqrt(jnp.sum(out * out, axis=1, keepdims=True)),
                      1e-12)
    out = out / nrm
    mean = jnp.mean(out, axis=0, keepdims=True)
    var = jnp.mean((out - mean) ** 2, axis=0, keepdims=True)
    out = (out - mean) / jnp.sqrt(var + 1e-5) * g_ref[...] + be_ref[...]
    out_ref[...] = jnp.maximum(out, 0.0)


_combine_call = pl.pallas_call(
    _combine_body, out_shape=jax.ShapeDtypeStruct((N, D), jnp.float32),
    compiler_params=_TC_PARAMS)


def _heads_body(h_ref, wo1_ref, bo1_ref, wo2_ref, bo2_ref, wc1_ref, bc1_ref,
                wc2_ref, bc2_ref, wb1_ref, bb1_ref, wb2_ref, bb2_ref,
                order_ref, cost_ref, bull_ref):
    h = h_ref[...]
    order_ref[...] = _dot(jnp.maximum(_dot(h, wo1_ref[...]) + bo1_ref[...],
                                      0.0), wo2_ref[...]) + bo2_ref[...]
    cost_ref[...] = _dot(jnp.maximum(_dot(h, wc1_ref[...]) + bc1_ref[...],
                                     0.0), wc2_ref[...]) + bc2_ref[...]
    bull_ref[...] = _dot(jnp.maximum(_dot(h, wb1_ref[...]) + bb1_ref[...],
                                     0.0), wb2_ref[...]) + bb2_ref[...]


_heads_call = pl.pallas_call(
    _heads_body,
    out_shape=(jax.ShapeDtypeStruct((N, 1), jnp.float32),
               jax.ShapeDtypeStruct((N, 1), jnp.float32),
               jax.ShapeDtypeStruct((N, 1), jnp.float32)),
    compiler_params=_TC_PARAMS)


def kernel(x, edge_index, params):
    p = params
    pad = E_PAD - E
    src_p = jnp.concatenate(
        [edge_index[0], jnp.zeros((pad,), jnp.int32)])
    dst_p = jnp.concatenate(
        [edge_index[1], jnp.full((pad,), N_PAD - 1, jnp.int32)])

    cnt_parts = _make_sc_count()(dst_p)
    inv = _inv_call(cnt_parts)
    h = _enc_call(x, p['W_enc'], p['b_enc'])
    for l in range(3):
        agg_parts = _make_sc_agg()(h, src_p, dst_p)
        h = _combine_call(agg_parts, inv, h, p['W_l%d' % l], p['b_l%d' % l],
                          p['W_r%d' % l], p['bn_g%d' % l], p['bn_b%d' % l])
    order, cost, bull = _heads_call(
        h, p['W_o1'], p['b_o1'], p['W_o2'], p['b_o2'],
        p['W_c1'], p['b_c1'], p['W_c2'], p['b_c2'],
        p['W_b1'], p['b_b1'], p['W_b2'], p['b_b2'])
    return order, cost, bull, h
